# Initial kernel scaffold; baseline (speedup 1.0000x reference)
import functools

import jax
import jax.numpy as jnp
from jax import lax
from jax.experimental import pallas as pl
from jax.experimental.pallas import tpu as pltpu
from jax.experimental.pallas import tpu_sc as plsc

N = 10000
E = 320000
D = 128
NP = 10240
NW = 32
EPW = E // NW
CW = 128
CH = (EPW + CW - 1) // CW
EPP = CH * CW
ROWS_PER_TILE = NP // 16


def _mm_body(x_ref, ws_ref, wl_ref, bl_ref, wd_ref, attd_ref, atts_ref,
             h_ref, skip_ref, as_ref, ad_ref, mas_ref, mad_ref):
    i = pl.program_id(0)
    xb = x_ref[...]
    h = jnp.dot(xb, ws_ref[...], preferred_element_type=jnp.float32)
    h_ref[...] = h
    skip_ref[...] = (jnp.dot(xb, wl_ref[...], preferred_element_type=jnp.float32)
                     + bl_ref[...][None, :])
    a_s = jnp.sum(h * atts_ref[...][None, :], axis=1)
    as_ref[...] = a_s
    wdv = jnp.sum(wd_ref[...] * attd_ref[...][None, :], axis=1)
    a_d = jnp.sum(xb * wdv[None, :], axis=1)
    ad_ref[...] = a_d

    @pl.when(i == 0)
    def _():
        mas_ref[0, 0] = -jnp.inf
        mad_ref[0, 0] = -jnp.inf

    mas_ref[0, 0] = jnp.maximum(mas_ref[0, 0], jnp.max(a_s))
    mad_ref[0, 0] = jnp.maximum(mad_ref[0, 0], jnp.max(a_d))


def _mm(x, w_src, wl, bl, w_dst, att_dst, att_src):
    blk = 512
    grid = NP // blk
    return pl.pallas_call(
        _mm_body,
        grid=(grid,),
        in_specs=[
            pl.BlockSpec((blk, D), lambda i: (i, 0)),
            pl.BlockSpec((D, D), lambda i: (0, 0)),
            pl.BlockSpec((D, D), lambda i: (0, 0)),
            pl.BlockSpec((D,), lambda i: (0,)),
            pl.BlockSpec((D, D), lambda i: (0, 0)),
            pl.BlockSpec((D,), lambda i: (0,)),
            pl.BlockSpec((D,), lambda i: (0,)),
        ],
        out_specs=[
            pl.BlockSpec((blk, D), lambda i: (i, 0)),
            pl.BlockSpec((blk, D), lambda i: (i, 0)),
            pl.BlockSpec((blk,), lambda i: (i,)),
            pl.BlockSpec((blk,), lambda i: (i,)),
            pl.BlockSpec((1, 1), lambda i: (0, 0), memory_space=pltpu.SMEM),
            pl.BlockSpec((1, 1), lambda i: (0, 0), memory_space=pltpu.SMEM),
        ],
        out_shape=[
            jax.ShapeDtypeStruct((NP, D), jnp.float32),
            jax.ShapeDtypeStruct((NP, D), jnp.float32),
            jax.ShapeDtypeStruct((NP,), jnp.float32),
            jax.ShapeDtypeStruct((NP,), jnp.float32),
            jax.ShapeDtypeStruct((1, 1), jnp.float32),
            jax.ShapeDtypeStruct((1, 1), jnp.float32),
        ],
    )(x, w_src, wl, bl, w_dst, att_dst, att_src)


def _sc_edge_body(h_hbm, as_hbm, ad_hbm, src_hbm, dst_hbm, c_hbm,
                  zr_hbm, zd_hbm, acc_out, den_out,
                  acc_sh, den_sh, asv, adv, srcv, dstv, pbuf, rows, cv, sem):
    ci = lax.axis_index("c")
    si = lax.axis_index("s")
    wid = ci * 16 + si

    pltpu.sync_copy(zr_hbm, acc_sh.at[pl.ds(si * ROWS_PER_TILE, ROWS_PER_TILE)])
    pltpu.sync_copy(zd_hbm, den_sh.at[pl.ds(si * ROWS_PER_TILE, ROWS_PER_TILE)])
    pltpu.sync_copy(as_hbm, asv)
    pltpu.sync_copy(ad_hbm, adv)
    pltpu.sync_copy(src_hbm.at[wid], srcv)
    pltpu.sync_copy(dst_hbm.at[wid], dstv)
    pltpu.sync_copy(c_hbm, cv)
    plsc.subcore_barrier()

    cvec = cv[...]

    def chunk_body(i, carry):
        pltpu.async_copy(h_hbm.at[srcv.at[i]], rows, sem).wait()

        def vreg_body(k, c2):
            sidx = srcv[i, pl.ds(k * 16, 16)]
            didx = dstv[i, pl.ds(k * 16, 16)]
            e = plsc.load_gather(asv, [sidx]) + plsc.load_gather(adv, [didx])
            e = jnp.where(e > 0, e, 0.2 * e)
            pbuf[pl.ds(k * 16, 16)] = jnp.exp(e - cvec)
            return c2

        lax.fori_loop(0, CW // 16, vreg_body, 0, unroll=True)

        pltpu.sync_copy(pbuf, den_sh.at[dstv.at[i]], add=True)

        def row_body(j, c3):
            ps = pbuf[j]
            for k2 in range(8):
                rows[j, pl.ds(k2 * 16, 16)] = rows[j, pl.ds(k2 * 16, 16)] * ps
            return c3

        lax.fori_loop(0, CW, row_body, 0)

        pltpu.sync_copy(rows, acc_sh.at[dstv.at[i]], add=True)
        return carry

    lax.fori_loop(0, CH, chunk_body, 0)
    plsc.subcore_barrier()

    sl = pl.ds(si * ROWS_PER_TILE, ROWS_PER_TILE)
    pltpu.sync_copy(acc_sh.at[sl], acc_out.at[ci, sl])
    pltpu.sync_copy(den_sh.at[sl], den_out.at[ci, sl])


@functools.partial(
    pl.kernel,
    out_type=(
        jax.ShapeDtypeStruct((2, NP, D), jnp.float32),
        jax.ShapeDtypeStruct((2, NP), jnp.float32),
    ),
    mesh=plsc.VectorSubcoreMesh(core_axis_name="c", subcore_axis_name="s"),
    scratch_types=[
        pltpu.VMEM_SHARED((NP, D), jnp.float32),
        pltpu.VMEM_SHARED((NP,), jnp.float32),
        pltpu.VMEM((NP,), jnp.float32),
        pltpu.VMEM((NP,), jnp.float32),
        pltpu.VMEM((CH, CW), jnp.int32),
        pltpu.VMEM((CH, CW), jnp.int32),
        pltpu.VMEM((CW,), jnp.float32),
        pltpu.VMEM((CW, D), jnp.float32),
        pltpu.VMEM((16,), jnp.float32),
        pltpu.SemaphoreType.DMA,
    ],
)
def _sc_edge(*refs):
    _sc_edge_body(*refs)


def _comb_body(acc_ref, den_ref, skip_ref, b_ref, out_ref):
    num = acc_ref[0, :, :] + acc_ref[1, :, :]
    dn = den_ref[0, :] + den_ref[1, :] + 1e-16
    h = num / dn[:, None] + skip_ref[...] + b_ref[...][None, :]
    out_ref[...] = jnp.maximum(h, 0.0)


def _comb(acc, den, skip, b):
    blk = 512
    return pl.pallas_call(
        _comb_body,
        grid=(NP // blk,),
        in_specs=[
            pl.BlockSpec((2, blk, D), lambda i: (0, i, 0)),
            pl.BlockSpec((2, blk), lambda i: (0, i)),
            pl.BlockSpec((blk, D), lambda i: (i, 0)),
            pl.BlockSpec((D,), lambda i: (0,)),
        ],
        out_specs=pl.BlockSpec((blk, D), lambda i: (i, 0)),
        out_shape=jax.ShapeDtypeStruct((NP, D), jnp.float32),
    )(acc, den, skip, b)


def _layer(x_pad, src3, dst3, zr, zd, w_src, w_dst, att_src, att_dst, b, wl, bl):
    h, skip, a_s, a_d, mas, mad = _mm(x_pad, w_src, wl, bl, w_dst, att_dst, att_src)
    cb = mas[0, 0] + mad[0, 0]
    c = jnp.where(cb > 0, cb, 0.2 * cb)
    cvec = jnp.full((16,), c, jnp.float32)
    acc, den = _sc_edge(h, a_s, a_d, src3, dst3, cvec, zr, zd)
    return _comb(acc, den, skip, b)


def kernel(x, edge_index, W1_src, W1_dst, att1_src, att1_dst, b1, Wl1, bl1,
           W2_src, W2_dst, att2_src, att2_dst, b2, Wl2, bl2):
    x_pad = jnp.pad(x, ((0, NP - N), (0, 0)))
    src = edge_index[0].astype(jnp.int32).reshape(NW, EPW)
    dst = edge_index[1].astype(jnp.int32).reshape(NW, EPW)
    pad = ((0, 0), (0, EPP - EPW))
    src3 = jnp.pad(src, pad, constant_values=NP - 1).reshape(NW, CH, CW)
    dst3 = jnp.pad(dst, pad, constant_values=NP - 1).reshape(NW, CH, CW)
    zr = jnp.zeros((ROWS_PER_TILE, D), jnp.float32)
    zd = jnp.zeros((ROWS_PER_TILE,), jnp.float32)

    h = _layer(x_pad, src3, dst3, zr, zd,
               W1_src, W1_dst, att1_src, att1_dst, b1, Wl1, bl1)
    out = _layer(h, src3, dst3, zr, zd,
                 W2_src, W2_dst, att2_src, att2_dst, b2, Wl2, bl2)
    return out[:N]


# speedup vs baseline: 21.2467x; 21.2467x over previous
import functools

import jax
import jax.numpy as jnp
from jax import lax
from jax.experimental import pallas as pl
from jax.experimental.pallas import tpu as pltpu
from jax.experimental.pallas import tpu_sc as plsc

N = 10000
E = 320000
D = 128
NP = 10240
NW = 32
EPW = E // NW
CW = 128
CH = (EPW + CW - 1) // CW
EPP = CH * CW
ROWS_PER_TILE = NP // 16


def _mm_body(x_ref, ws_ref, wl_ref, bl_ref, wd_ref, attd_ref, atts_ref,
             h_ref, skip_ref, as_ref, ad_ref, mas_ref, mad_ref):
    i = pl.program_id(0)
    xb = x_ref[...]
    h = jnp.dot(xb, ws_ref[...], preferred_element_type=jnp.float32)
    h_ref[...] = h
    skip_ref[...] = (jnp.dot(xb, wl_ref[...], preferred_element_type=jnp.float32)
                     + bl_ref[...][None, :])
    a_s = jnp.sum(h * atts_ref[...][None, :], axis=1)
    as_ref[...] = a_s
    wdv = jnp.sum(wd_ref[...] * attd_ref[...][None, :], axis=1)
    a_d = jnp.sum(xb * wdv[None, :], axis=1)
    ad_ref[...] = a_d

    @pl.when(i == 0)
    def _():
        mas_ref[0, 0] = -jnp.inf
        mad_ref[0, 0] = -jnp.inf

    mas_ref[0, 0] = jnp.maximum(mas_ref[0, 0], jnp.max(a_s))
    mad_ref[0, 0] = jnp.maximum(mad_ref[0, 0], jnp.max(a_d))


def _mm(x, w_src, wl, bl, w_dst, att_dst, att_src):
    blk = 512
    grid = NP // blk
    return pl.pallas_call(
        _mm_body,
        grid=(grid,),
        in_specs=[
            pl.BlockSpec((blk, D), lambda i: (i, 0)),
            pl.BlockSpec((D, D), lambda i: (0, 0)),
            pl.BlockSpec((D, D), lambda i: (0, 0)),
            pl.BlockSpec((D,), lambda i: (0,)),
            pl.BlockSpec((D, D), lambda i: (0, 0)),
            pl.BlockSpec((D,), lambda i: (0,)),
            pl.BlockSpec((D,), lambda i: (0,)),
        ],
        out_specs=[
            pl.BlockSpec((blk, D), lambda i: (i, 0)),
            pl.BlockSpec((blk, D), lambda i: (i, 0)),
            pl.BlockSpec((blk,), lambda i: (i,)),
            pl.BlockSpec((blk,), lambda i: (i,)),
            pl.BlockSpec((1, 1), lambda i: (0, 0), memory_space=pltpu.SMEM),
            pl.BlockSpec((1, 1), lambda i: (0, 0), memory_space=pltpu.SMEM),
        ],
        out_shape=[
            jax.ShapeDtypeStruct((NP, D), jnp.float32),
            jax.ShapeDtypeStruct((NP, D), jnp.float32),
            jax.ShapeDtypeStruct((NP,), jnp.float32),
            jax.ShapeDtypeStruct((NP,), jnp.float32),
            jax.ShapeDtypeStruct((1, 1), jnp.float32),
            jax.ShapeDtypeStruct((1, 1), jnp.float32),
        ],
    )(x, w_src, wl, bl, w_dst, att_dst, att_src)


def _sc_edge_body(h_hbm, as_hbm, ad_hbm, src_hbm, dst_hbm, c_hbm,
                  zr_hbm, zd_hbm, acc_out, den_out,
                  acc_sh, den_sh, asv, adv, srcv, dstv, pbuf, rows, cv, sem):
    ci = lax.axis_index("c")
    si = lax.axis_index("s")
    wid = ci * 16 + si

    pltpu.sync_copy(zr_hbm, acc_sh.at[pl.ds(si * ROWS_PER_TILE, ROWS_PER_TILE)])
    pltpu.sync_copy(zd_hbm, den_sh.at[pl.ds(si * ROWS_PER_TILE, ROWS_PER_TILE)])
    pltpu.sync_copy(as_hbm, asv)
    pltpu.sync_copy(ad_hbm, adv)
    pltpu.sync_copy(c_hbm, cv)
    plsc.subcore_barrier()

    cvec = cv[...]

    def chunk_body(i, carry):
        pltpu.sync_copy(src_hbm.at[wid, i], srcv)
        pltpu.sync_copy(dst_hbm.at[wid, i], dstv)
        pltpu.async_copy(h_hbm.at[srcv], rows, sem).wait()

        def vreg_body(k, c2):
            sidx = srcv[pl.ds(k * 16, 16)]
            didx = dstv[pl.ds(k * 16, 16)]
            e = plsc.load_gather(asv, [sidx]) + plsc.load_gather(adv, [didx])
            e = jnp.where(e > 0, e, 0.2 * e)
            pbuf[pl.ds(k * 16, 16)] = jnp.exp(e - cvec)
            return c2

        lax.fori_loop(0, CW // 16, vreg_body, 0, unroll=True)

        pltpu.sync_copy(pbuf, den_sh.at[dstv], add=True)

        def row_body(g, c3):
            pv = pbuf[pl.ds(g * 16, 16)]
            for l in range(16):
                ps = pv[l]
                j = g * 16 + l
                for k2 in range(8):
                    rows[j, pl.ds(k2 * 16, 16)] = rows[j, pl.ds(k2 * 16, 16)] * ps
            return c3

        lax.fori_loop(0, CW // 16, row_body, 0)

        pltpu.sync_copy(rows, acc_sh.at[dstv], add=True)
        return carry

    lax.fori_loop(0, CH, chunk_body, 0)
    plsc.subcore_barrier()

    sl = pl.ds(si * ROWS_PER_TILE, ROWS_PER_TILE)
    pltpu.sync_copy(acc_sh.at[sl], acc_out.at[ci, sl])
    pltpu.sync_copy(den_sh.at[sl], den_out.at[ci, sl])


@functools.partial(
    pl.kernel,
    out_type=(
        jax.ShapeDtypeStruct((2, NP, D), jnp.float32),
        jax.ShapeDtypeStruct((2, NP), jnp.float32),
    ),
    mesh=plsc.VectorSubcoreMesh(core_axis_name="c", subcore_axis_name="s"),
    compiler_params=pltpu.CompilerParams(needs_layout_passes=False),
    scratch_types=[
        pltpu.VMEM_SHARED((NP, D), jnp.float32),
        pltpu.VMEM_SHARED((NP,), jnp.float32),
        pltpu.VMEM((NP,), jnp.float32),
        pltpu.VMEM((NP,), jnp.float32),
        pltpu.VMEM((CW,), jnp.int32),
        pltpu.VMEM((CW,), jnp.int32),
        pltpu.VMEM((CW,), jnp.float32),
        pltpu.VMEM((CW, D), jnp.float32),
        pltpu.VMEM((16,), jnp.float32),
        pltpu.SemaphoreType.DMA,
    ],
)
def _sc_edge(*refs):
    _sc_edge_body(*refs)


def _comb_body(acc_ref, den_ref, skip_ref, b_ref, out_ref):
    num = acc_ref[0, :, :] + acc_ref[1, :, :]
    dn = den_ref[0, :] + den_ref[1, :] + 1e-16
    h = num / dn[:, None] + skip_ref[...] + b_ref[...][None, :]
    out_ref[...] = jnp.maximum(h, 0.0)


def _comb(acc, den, skip, b):
    blk = 512
    return pl.pallas_call(
        _comb_body,
        grid=(NP // blk,),
        in_specs=[
            pl.BlockSpec((2, blk, D), lambda i: (0, i, 0)),
            pl.BlockSpec((2, blk), lambda i: (0, i)),
            pl.BlockSpec((blk, D), lambda i: (i, 0)),
            pl.BlockSpec((D,), lambda i: (0,)),
        ],
        out_specs=pl.BlockSpec((blk, D), lambda i: (i, 0)),
        out_shape=jax.ShapeDtypeStruct((NP, D), jnp.float32),
    )(acc, den, skip, b)


def _layer(x_pad, src3, dst3, zr, zd, w_src, w_dst, att_src, att_dst, b, wl, bl):
    h, skip, a_s, a_d, mas, mad = _mm(x_pad, w_src, wl, bl, w_dst, att_dst, att_src)
    cb = mas[0, 0] + mad[0, 0]
    c = jnp.where(cb > 0, cb, 0.2 * cb)
    cvec = jnp.full((16,), c, jnp.float32)
    acc, den = _sc_edge(h, a_s, a_d, src3, dst3, cvec, zr, zd)
    return _comb(acc, den, skip, b)


def kernel(x, edge_index, W1_src, W1_dst, att1_src, att1_dst, b1, Wl1, bl1,
           W2_src, W2_dst, att2_src, att2_dst, b2, Wl2, bl2):
    x_pad = jnp.pad(x, ((0, NP - N), (0, 0)))
    src = edge_index[0].astype(jnp.int32).reshape(NW, EPW)
    dst = edge_index[1].astype(jnp.int32).reshape(NW, EPW)
    pad = ((0, 0), (0, EPP - EPW))
    src3 = jnp.pad(src, pad, constant_values=NP - 1).reshape(NW, CH, CW)
    dst3 = jnp.pad(dst, pad, constant_values=NP - 1).reshape(NW, CH, CW)
    zr = jnp.zeros((ROWS_PER_TILE, D), jnp.float32)
    zd = jnp.zeros((ROWS_PER_TILE,), jnp.float32)

    h = _layer(x_pad, src3, dst3, zr, zd,
               W1_src, W1_dst, att1_src, att1_dst, b1, Wl1, bl1)
    out = _layer(h, src3, dst3, zr, zd,
                 W2_src, W2_dst, att2_src, att2_dst, b2, Wl2, bl2)
    return out[:N]
